# baseline (speedup 1.0000x reference)
import functools

import jax
import jax.numpy as jnp
from jax import lax
from jax.experimental import pallas as pl
from jax.experimental.pallas import tpu as pltpu
from jax.experimental.pallas import tpu_sc as plsc

N_ATOMS = 10000
N_PAD = 10240
E_ATOMS = 320000
N_FRAGS = 2000
E_FRAG = 8000
D = 128

NC = 2
NS = 16
NW = NC * NS
L = 16

EC_CHUNKS = 80
EPW = EC_CHUNKS * 128
E_PAD = NW * EPW
JUNK = 10008
Z_ROWS = 10112
Z_STRIPE = Z_ROWS // NS

A_CHUNKS = N_PAD // NS // 128
F_ROWS = 2048
F_STRIPE = F_ROWS // NS
JUNK_F = 2040
EF_PAD = 8192
EF_CHUNKS = EF_PAD // NS // 128

_MESH = plsc.VectorSubcoreMesh(core_axis_name="c", subcore_axis_name="s")
_SC_PARAMS = pltpu.CompilerParams(needs_layout_passes=False)


def _copy_idx(dst_ref, src_ref, base):
    for c in range(8):
        dst_ref[pl.ds(c * L, L)] = src_ref[pl.ds(base + c * L, L)]


@functools.partial(
    pl.kernel,
    out_type=jax.ShapeDtypeStruct((NW * N_PAD,), jnp.float32),
    mesh=_MESH,
    compiler_params=_SC_PARAMS,
    scratch_types=[
        pltpu.VMEM((EPW,), jnp.int32),
        pltpu.VMEM((N_PAD,), jnp.float32),
    ],
)
def _sc_hist(src_hbm, counts_hbm, src_v, counts_v):
    wid = lax.axis_index("s") * NC + lax.axis_index("c")
    pltpu.sync_copy(src_hbm.at[pl.ds(wid * EPW, EPW)], src_v)

    zeros = jnp.zeros((L,), jnp.float32)

    def zero_body(i, _):
        counts_v[pl.ds(i * L, L)] = zeros
        return 0

    lax.fori_loop(0, N_PAD // L, zero_body, 0)

    ones = jnp.ones((L,), jnp.float32)

    def hist_body(i, _):
        idx = src_v[pl.ds(i * L, L)]
        plsc.addupdate_scatter(counts_v, [idx], ones)
        return 0

    lax.fori_loop(0, EPW // L, hist_body, 0)
    pltpu.sync_copy(counts_v, counts_hbm.at[pl.ds(wid * N_PAD, N_PAD)])


def _tc_emb_body(x_ref, w_ref, b_ref, cnt_ref, y_ref, y16_ref, dinv_ref):
    deg = jnp.sum(cnt_ref[...], axis=1, keepdims=True) + 1.0
    dinv = lax.rsqrt(deg)
    dinv_ref[...] = dinv
    xw = jnp.dot(x_ref[...], w_ref[...], preferred_element_type=jnp.float32)
    yv = (xw + b_ref[...]) * dinv[:N_ATOMS]
    y_ref[:N_ATOMS] = yv
    y_ref[N_ATOMS:] = jnp.zeros((N_PAD - N_ATOMS, D), jnp.float32)
    y16_ref[:N_ATOMS] = yv.astype(jnp.bfloat16)
    y16_ref[N_ATOMS:] = jnp.zeros((N_PAD - N_ATOMS, D), jnp.bfloat16)


_tc_emb = pl.pallas_call(
    _tc_emb_body,
    out_shape=(
        jax.ShapeDtypeStruct((N_PAD, D), jnp.float32),
        jax.ShapeDtypeStruct((N_PAD, D), jnp.bfloat16),
        jax.ShapeDtypeStruct((N_PAD, 1), jnp.float32),
    ),
)


@functools.partial(
    pl.kernel,
    out_type=jax.ShapeDtypeStruct((NC * 2, Z_ROWS, D), jnp.bfloat16),
    mesh=_MESH,
    compiler_params=pltpu.CompilerParams(needs_layout_passes=False,
                                         use_tc_tiling_on_sc=False),
    scratch_types=[
        pltpu.VMEM((EPW // 2,), jnp.int32),
        pltpu.VMEM((EPW // 2,), jnp.int32),
        pltpu.VMEM((128,), jnp.int32),
        pltpu.VMEM((128,), jnp.int32),
        pltpu.VMEM((128,), jnp.int32),
        pltpu.VMEM((128,), jnp.int32),
        pltpu.VMEM((128, D), jnp.bfloat16),
        pltpu.VMEM((128, D), jnp.bfloat16),
        pltpu.VMEM((128, D), jnp.bfloat16),
        pltpu.VMEM((128, D), jnp.bfloat16),
        pltpu.SemaphoreType.DMA,
        pltpu.SemaphoreType.DMA,
        pltpu.SemaphoreType.DMA,
        pltpu.SemaphoreType.DMA,
        pltpu.SemaphoreType.DMA,
        pltpu.SemaphoreType.DMA,
        pltpu.SemaphoreType.DMA,
        pltpu.SemaphoreType.DMA,
        pltpu.VMEM_SHARED((Z_ROWS, D), jnp.bfloat16),
        pltpu.VMEM_SHARED((Z_ROWS, D), jnp.bfloat16),
    ],
)
def _sc_edges(y_hbm, srcp_hbm, tgtp_hbm, zeros_hbm, zpart_hbm,
              src_v, tgt_v, tcur0_v, tcur1_v, tcur2_v, tcur3_v,
              rows0_v, rows1_v, rows2_v, rows3_v,
              gsem0, gsem1, gsem2, gsem3, ssem0, ssem1, ssem2, ssem3,
              zacc, zacc2):
    cid = lax.axis_index("c")
    sid = lax.axis_index("s")
    wid = sid * NC + cid

    pltpu.sync_copy(zeros_hbm,
                    zacc.at[pl.ds(sid * Z_STRIPE, Z_STRIPE)])
    pltpu.sync_copy(zeros_hbm,
                    zacc2.at[pl.ds(sid * Z_STRIPE, Z_STRIPE)])
    plsc.subcore_barrier()

    rows = (rows0_v, rows1_v, rows2_v, rows3_v)
    gsems = (gsem0, gsem1, gsem2, gsem3)
    ssems = (ssem0, ssem1, ssem2, ssem3)
    tcurs = (tcur0_v, tcur1_v, tcur2_v, tcur3_v)
    accs = (zacc, zacc2, zacc, zacc2)

    def _gwait(b):
        pltpu.make_async_copy(y_hbm.at[src_v.at[pl.ds(0, 128)]],
                              rows[b], gsems[b]).wait()

    def _swait(b):
        pltpu.make_async_copy(rows[b], accs[b].at[tcurs[b]], ssems[b]).wait()

    half = EC_CHUNKS // 2
    for p in range(2):
        base = wid * EPW + p * (EPW // 2)
        pltpu.sync_copy(srcp_hbm.at[pl.ds(base, EPW // 2)], src_v)
        pltpu.sync_copy(tgtp_hbm.at[pl.ds(base, EPW // 2)], tgt_v)

        pltpu.async_copy(y_hbm.at[src_v.at[pl.ds(0, 128)]], rows0_v, gsem0)
        pltpu.async_copy(y_hbm.at[src_v.at[pl.ds(128, 128)]], rows1_v, gsem1)

        def chunk_body(g, _):
            for b in range(4):
                j = 4 * g + b
                _gwait(b)
                _copy_idx(tcurs[b], tgt_v, j * 128)
                pltpu.async_copy(rows[b], accs[b].at[tcurs[b]], ssems[b],
                                 add=True)
                b2 = (b + 2) % 4
                if b < 2:
                    @pl.when(g >= 1)
                    def _():
                        _swait(b2)
                    pltpu.async_copy(
                        y_hbm.at[src_v.at[pl.ds((j + 2) * 128, 128)]],
                        rows[b2], gsems[b2])
                else:
                    @pl.when(g < half // 4 - 1)
                    def _():
                        _swait(b2)
                        pltpu.async_copy(
                            y_hbm.at[src_v.at[pl.ds((j + 2) * 128, 128)]],
                            rows[b2], gsems[b2])
            return 0

        lax.fori_loop(0, half // 4, chunk_body, 0)
        for b in range(4):
            _swait(b)
    plsc.subcore_barrier()
    pltpu.sync_copy(zacc.at[pl.ds(sid * Z_STRIPE, Z_STRIPE)],
                    zpart_hbm.at[cid * 2, pl.ds(sid * Z_STRIPE, Z_STRIPE)])
    pltpu.sync_copy(zacc2.at[pl.ds(sid * Z_STRIPE, Z_STRIPE)],
                    zpart_hbm.at[cid * 2 + 1, pl.ds(sid * Z_STRIPE, Z_STRIPE)])


def _tc_fin_body(zpart_ref, y_ref, dinv_ref, out_ref):
    z = (zpart_ref[0, :N_ATOMS, :].astype(jnp.float32)
         + zpart_ref[1, :N_ATOMS, :].astype(jnp.float32)
         + zpart_ref[2, :N_ATOMS, :].astype(jnp.float32)
         + zpart_ref[3, :N_ATOMS, :].astype(jnp.float32))
    out_ref[...] = dinv_ref[:N_ATOMS] * (z + y_ref[:N_ATOMS])


_tc_fin = pl.pallas_call(
    _tc_fin_body,
    out_shape=jax.ShapeDtypeStruct((N_ATOMS, D), jnp.float32),
)


@functools.partial(
    pl.kernel,
    out_type=jax.ShapeDtypeStruct((F_ROWS, D), jnp.float32),
    mesh=_MESH,
    compiler_params=_SC_PARAMS,
    scratch_types=[
        pltpu.VMEM((A_CHUNKS * 128,), jnp.int32),
        pltpu.VMEM((A_CHUNKS * 128,), jnp.int32),
        pltpu.VMEM((EF_CHUNKS * 128,), jnp.int32),
        pltpu.VMEM((EF_CHUNKS * 128,), jnp.int32),
        pltpu.VMEM((128,), jnp.int32),
        pltpu.VMEM((128,), jnp.int32),
        pltpu.VMEM((128, D), jnp.float32),
        pltpu.VMEM((128, D), jnp.float32),
        pltpu.SemaphoreType.DMA,
        pltpu.SemaphoreType.DMA,
        pltpu.VMEM_SHARED((F_ROWS, D), jnp.float32),
        pltpu.VMEM_SHARED((F_ROWS, D), jnp.float32),
    ],
)
def _sc_frag(xnew_hbm, aid_hbm, a2f_hbm, fsrc_hbm, ftgt_hbm, zeros_hbm, ffs_hbm,
             aid_v, a2f_v, fsrc_v, ftgt_v, icur0_v, icur1_v, rows0_v, rows1_v,
             gsem0, gsem1, fragacc, ffs_acc):
    cid = lax.axis_index("c")
    sid = lax.axis_index("s")

    @pl.when(cid == 0)
    def _():
        napw = A_CHUNKS * 128
        nefw = EF_CHUNKS * 128
        pltpu.sync_copy(zeros_hbm.at[pl.ds(0, F_STRIPE)],
                        fragacc.at[pl.ds(sid * F_STRIPE, F_STRIPE)])
        pltpu.sync_copy(zeros_hbm.at[pl.ds(0, F_STRIPE)],
                        ffs_acc.at[pl.ds(sid * F_STRIPE, F_STRIPE)])
        pltpu.sync_copy(aid_hbm.at[pl.ds(sid * napw, napw)], aid_v)
        pltpu.sync_copy(a2f_hbm.at[pl.ds(sid * napw, napw)], a2f_v)
        pltpu.sync_copy(fsrc_hbm.at[pl.ds(sid * nefw, nefw)], fsrc_v)
        pltpu.sync_copy(ftgt_hbm.at[pl.ds(sid * nefw, nefw)], ftgt_v)
        plsc.subcore_barrier()

        bufs = ((rows0_v, gsem0, icur0_v), (rows1_v, gsem1, icur1_v))

        def _pipe(n_chunks, gather_src, gidx_v, sidx_v, acc):
            for b in range(2):
                pltpu.async_copy(
                    gather_src.at[gidx_v.at[pl.ds(b * 128, 128)]],
                    bufs[b][0], bufs[b][1])
            for j in range(n_chunks):
                rows_v, gsem, icur_v = bufs[j % 2]
                pltpu.make_async_copy(
                    gather_src.at[gidx_v.at[pl.ds(0, 128)]],
                    rows_v, gsem).wait()
                _copy_idx(icur_v, sidx_v, j * 128)
                pltpu.sync_copy(rows_v, acc.at[icur_v], add=True)
                if j + 2 < n_chunks:
                    pltpu.async_copy(
                        gather_src.at[gidx_v.at[pl.ds((j + 2) * 128, 128)]],
                        rows_v, gsem)

        _pipe(A_CHUNKS, xnew_hbm, aid_v, a2f_v, fragacc)
        plsc.subcore_barrier()

        _pipe(EF_CHUNKS, fragacc, fsrc_v, ftgt_v, ffs_acc)
        plsc.subcore_barrier()

        pltpu.sync_copy(ffs_acc.at[pl.ds(sid * F_STRIPE, F_STRIPE)],
                        ffs_hbm.at[pl.ds(sid * F_STRIPE, F_STRIPE)])


def _tc_mlp_body(ffs_ref, w1_ref, b1_ref, w2_ref, b2_ref, out_ref):
    h = jnp.dot(ffs_ref[:N_FRAGS], w1_ref[...],
                preferred_element_type=jnp.float32) + b1_ref[...]
    h = jnp.maximum(h, 0.0)
    out_ref[...] = jnp.dot(h, w2_ref[...],
                           preferred_element_type=jnp.float32) + b2_ref[...]


_tc_mlp = pl.pallas_call(
    _tc_mlp_body,
    out_shape=jax.ShapeDtypeStruct((N_FRAGS, D), jnp.float32),
)


def kernel(x_atoms, edge_index, edge_attr, frag_index, x_frags, atom_to_frag_ids,
           node_feautures_bond_graph, edge_index_bonds_graph, edge_attr_bond_graph,
           W_atom, b_atom, W_edge, b_edge, W_proj, b_proj, a_b,
           W_frag1, b_frag1, W_frag2, b_frag2):
    src = edge_index[0]
    tgt = edge_index[1]
    pad_idx = jnp.arange(E_PAD - E_ATOMS, dtype=jnp.int32)
    srcp = jnp.concatenate([src, N_ATOMS + pad_idx % (N_PAD - N_ATOMS)])
    tgtp = jnp.concatenate([tgt, N_ATOMS + pad_idx % (Z_ROWS - N_ATOMS)])

    counts = _sc_hist(srcp)
    y, y16, dinv = _tc_emb(x_atoms, W_atom, b_atom.reshape(1, D),
                           counts.reshape(NW, N_PAD).T)

    zeros_stripe = jnp.zeros((Z_STRIPE, D), jnp.float32)
    zeros16 = jnp.zeros((Z_STRIPE, D), jnp.bfloat16)
    zpart = _sc_edges(y16, srcp, tgtp, zeros16)
    x_atoms_new = _tc_fin(zpart, y, dinv)

    aidp = jnp.arange(N_PAD, dtype=jnp.int32) % N_ATOMS
    a2fp = jnp.pad(atom_to_frag_ids, (0, N_PAD - N_ATOMS),
                   constant_values=JUNK_F)
    fsrcp = jnp.pad(frag_index[0], (0, EF_PAD - E_FRAG), constant_values=JUNK_F)
    ftgtp = jnp.pad(frag_index[1], (0, EF_PAD - E_FRAG), constant_values=JUNK_F)

    ffs = _sc_frag(x_atoms_new, aidp, a2fp, fsrcp, ftgtp, zeros_stripe)
    x_frags_new = _tc_mlp(ffs, W_frag1, b_frag1.reshape(1, 2 * D),
                          W_frag2, b_frag2.reshape(1, D))
    return (x_atoms_new, x_frags_new)


# speedup vs baseline: 1.1878x; 1.1878x over previous
import functools

import jax
import jax.numpy as jnp
from jax import lax
from jax.experimental import pallas as pl
from jax.experimental.pallas import tpu as pltpu
from jax.experimental.pallas import tpu_sc as plsc

N_ATOMS = 10000
N_PAD = 10240
E_ATOMS = 320000
N_FRAGS = 2000
E_FRAG = 8000
D = 128

NC = 2
NS = 16
NW = NC * NS
L = 16

EC_CHUNKS = 80
EPW = EC_CHUNKS * 128
E_PAD = NW * EPW
JUNK = 10008
Z_ROWS = 10112
Z_STRIPE = Z_ROWS // NS

A_CHUNKS = N_PAD // NS // 128
F_ROWS = 2048
F_STRIPE = F_ROWS // NS
JUNK_F = 2040
EF_PAD = 8192
EF_CHUNKS = EF_PAD // NS // 128

_MESH = plsc.VectorSubcoreMesh(core_axis_name="c", subcore_axis_name="s")
_SC_PARAMS = pltpu.CompilerParams(needs_layout_passes=False)


def _copy_idx(dst_ref, src_ref, base):
    for c in range(8):
        dst_ref[pl.ds(c * L, L)] = src_ref[pl.ds(base + c * L, L)]


@functools.partial(
    pl.kernel,
    out_type=jax.ShapeDtypeStruct((NW * N_PAD,), jnp.float32),
    mesh=_MESH,
    compiler_params=_SC_PARAMS,
    scratch_types=[
        pltpu.VMEM((EPW,), jnp.int32),
        pltpu.VMEM((N_PAD,), jnp.float32),
    ],
)
def _sc_hist(src_hbm, counts_hbm, src_v, counts_v):
    wid = lax.axis_index("s") * NC + lax.axis_index("c")
    pltpu.sync_copy(src_hbm.at[pl.ds(wid * EPW, EPW)], src_v)

    zeros = jnp.zeros((L,), jnp.float32)

    def zero_body(i, _):
        counts_v[pl.ds(i * L, L)] = zeros
        return 0

    lax.fori_loop(0, N_PAD // L, zero_body, 0)

    ones = jnp.ones((L,), jnp.float32)

    def hist_body(i, _):
        idx = src_v[pl.ds(i * L, L)]
        plsc.addupdate_scatter(counts_v, [idx], ones)
        return 0

    lax.fori_loop(0, EPW // L, hist_body, 0)
    pltpu.sync_copy(counts_v, counts_hbm.at[pl.ds(wid * N_PAD, N_PAD)])


def _tc_emb_body(x_ref, w_ref, b_ref, cnt_ref, y_ref, y16_ref, dinv_ref):
    deg = jnp.sum(cnt_ref[...], axis=1, keepdims=True) + 1.0
    dinv = lax.rsqrt(deg)
    dinv_ref[...] = dinv
    xw = jnp.dot(x_ref[...], w_ref[...], preferred_element_type=jnp.float32)
    yv = (xw + b_ref[...]) * dinv[:N_ATOMS]
    y_ref[:N_ATOMS] = yv
    y_ref[N_ATOMS:] = jnp.zeros((N_PAD - N_ATOMS, D), jnp.float32)
    y16_ref[:N_ATOMS] = yv.astype(jnp.bfloat16)
    y16_ref[N_ATOMS:] = jnp.zeros((N_PAD - N_ATOMS, D), jnp.bfloat16)


_tc_emb = pl.pallas_call(
    _tc_emb_body,
    out_shape=(
        jax.ShapeDtypeStruct((N_PAD, D), jnp.float32),
        jax.ShapeDtypeStruct((N_PAD, D), jnp.bfloat16),
        jax.ShapeDtypeStruct((N_PAD, 1), jnp.float32),
    ),
)


@functools.partial(
    pl.kernel,
    out_type=jax.ShapeDtypeStruct((NC, Z_ROWS, D), jnp.bfloat16),
    mesh=_MESH,
    compiler_params=pltpu.CompilerParams(needs_layout_passes=False,
                                         use_tc_tiling_on_sc=False),
    scratch_types=[
        pltpu.VMEM((EPW // 2,), jnp.int32),
        pltpu.VMEM((EPW // 2,), jnp.int32),
        pltpu.VMEM((128,), jnp.int32),
        pltpu.VMEM((128,), jnp.int32),
        pltpu.VMEM((128,), jnp.int32),
        pltpu.VMEM((128,), jnp.int32),
        pltpu.VMEM((128, D), jnp.bfloat16),
        pltpu.VMEM((128, D), jnp.bfloat16),
        pltpu.VMEM((128, D), jnp.bfloat16),
        pltpu.VMEM((128, D), jnp.bfloat16),
        pltpu.SemaphoreType.DMA,
        pltpu.SemaphoreType.DMA,
        pltpu.SemaphoreType.DMA,
        pltpu.SemaphoreType.DMA,
        pltpu.SemaphoreType.DMA,
        pltpu.SemaphoreType.DMA,
        pltpu.SemaphoreType.DMA,
        pltpu.SemaphoreType.DMA,
        pltpu.VMEM_SHARED((Z_ROWS, D), jnp.bfloat16),
    ],
)
def _sc_edges(y_hbm, srcp_hbm, tgtp_hbm, zeros_hbm, zpart_hbm,
              src_v, tgt_v, tcur0_v, tcur1_v, tcur2_v, tcur3_v,
              rows0_v, rows1_v, rows2_v, rows3_v,
              gsem0, gsem1, gsem2, gsem3, ssem0, ssem1, ssem2, ssem3,
              zacc):
    cid = lax.axis_index("c")
    sid = lax.axis_index("s")
    wid = sid * NC + cid

    pltpu.sync_copy(zeros_hbm,
                    zacc.at[pl.ds(sid * Z_STRIPE, Z_STRIPE)])
    plsc.subcore_barrier()

    rows = (rows0_v, rows1_v, rows2_v, rows3_v)
    gsems = (gsem0, gsem1, gsem2, gsem3)
    ssems = (ssem0, ssem1, ssem2, ssem3)
    tcurs = (tcur0_v, tcur1_v, tcur2_v, tcur3_v)
    accs = (zacc, zacc, zacc, zacc)

    def _gwait(b):
        pltpu.make_async_copy(y_hbm.at[src_v.at[pl.ds(0, 128)]],
                              rows[b], gsems[b]).wait()

    def _swait(b):
        pltpu.make_async_copy(rows[b], accs[b].at[tcurs[b]], ssems[b]).wait()

    half = EC_CHUNKS // 2
    for p in range(2):
        base = wid * EPW + p * (EPW // 2)
        pltpu.sync_copy(srcp_hbm.at[pl.ds(base, EPW // 2)], src_v)
        pltpu.sync_copy(tgtp_hbm.at[pl.ds(base, EPW // 2)], tgt_v)

        pltpu.async_copy(y_hbm.at[src_v.at[pl.ds(0, 128)]], rows0_v, gsem0)
        pltpu.async_copy(y_hbm.at[src_v.at[pl.ds(128, 128)]], rows1_v, gsem1)

        def chunk_body(g, _):
            for b in range(4):
                j = 4 * g + b
                _gwait(b)
                _copy_idx(tcurs[b], tgt_v, j * 128)
                pltpu.async_copy(rows[b], accs[b].at[tcurs[b]], ssems[b],
                                 add=True)
                b2 = (b + 2) % 4
                if b < 2:
                    @pl.when(g >= 1)
                    def _():
                        _swait(b2)
                    pltpu.async_copy(
                        y_hbm.at[src_v.at[pl.ds((j + 2) * 128, 128)]],
                        rows[b2], gsems[b2])
                else:
                    @pl.when(g < half // 4 - 1)
                    def _():
                        _swait(b2)
                        pltpu.async_copy(
                            y_hbm.at[src_v.at[pl.ds((j + 2) * 128, 128)]],
                            rows[b2], gsems[b2])
            return 0

        lax.fori_loop(0, half // 4, chunk_body, 0)
        for b in range(4):
            _swait(b)
    plsc.subcore_barrier()
    pltpu.sync_copy(zacc.at[pl.ds(sid * Z_STRIPE, Z_STRIPE)],
                    zpart_hbm.at[cid, pl.ds(sid * Z_STRIPE, Z_STRIPE)])


def _tc_fin_body(zpart_ref, y_ref, dinv_ref, out_ref):
    z = (zpart_ref[0, :N_ATOMS, :].astype(jnp.float32)
         + zpart_ref[1, :N_ATOMS, :].astype(jnp.float32))
    out_ref[...] = dinv_ref[:N_ATOMS] * (z + y_ref[:N_ATOMS])


_tc_fin = pl.pallas_call(
    _tc_fin_body,
    out_shape=jax.ShapeDtypeStruct((N_ATOMS, D), jnp.float32),
)


@functools.partial(
    pl.kernel,
    out_type=jax.ShapeDtypeStruct((F_ROWS, D), jnp.float32),
    mesh=_MESH,
    compiler_params=_SC_PARAMS,
    scratch_types=[
        pltpu.VMEM((A_CHUNKS * 128,), jnp.int32),
        pltpu.VMEM((A_CHUNKS * 128,), jnp.int32),
        pltpu.VMEM((EF_CHUNKS * 128,), jnp.int32),
        pltpu.VMEM((EF_CHUNKS * 128,), jnp.int32),
        pltpu.VMEM((128,), jnp.int32),
        pltpu.VMEM((128,), jnp.int32),
        pltpu.VMEM((128, D), jnp.float32),
        pltpu.VMEM((128, D), jnp.float32),
        pltpu.SemaphoreType.DMA,
        pltpu.SemaphoreType.DMA,
        pltpu.VMEM_SHARED((F_ROWS, D), jnp.float32),
        pltpu.VMEM_SHARED((F_ROWS, D), jnp.float32),
    ],
)
def _sc_frag(xnew_hbm, aid_hbm, a2f_hbm, fsrc_hbm, ftgt_hbm, zeros_hbm, ffs_hbm,
             aid_v, a2f_v, fsrc_v, ftgt_v, icur0_v, icur1_v, rows0_v, rows1_v,
             gsem0, gsem1, fragacc, ffs_acc):
    cid = lax.axis_index("c")
    sid = lax.axis_index("s")

    @pl.when(cid == 0)
    def _():
        napw = A_CHUNKS * 128
        nefw = EF_CHUNKS * 128
        pltpu.sync_copy(zeros_hbm.at[pl.ds(0, F_STRIPE)],
                        fragacc.at[pl.ds(sid * F_STRIPE, F_STRIPE)])
        pltpu.sync_copy(zeros_hbm.at[pl.ds(0, F_STRIPE)],
                        ffs_acc.at[pl.ds(sid * F_STRIPE, F_STRIPE)])
        pltpu.sync_copy(aid_hbm.at[pl.ds(sid * napw, napw)], aid_v)
        pltpu.sync_copy(a2f_hbm.at[pl.ds(sid * napw, napw)], a2f_v)
        pltpu.sync_copy(fsrc_hbm.at[pl.ds(sid * nefw, nefw)], fsrc_v)
        pltpu.sync_copy(ftgt_hbm.at[pl.ds(sid * nefw, nefw)], ftgt_v)
        plsc.subcore_barrier()

        bufs = ((rows0_v, gsem0, icur0_v), (rows1_v, gsem1, icur1_v))

        def _pipe(n_chunks, gather_src, gidx_v, sidx_v, acc):
            for b in range(2):
                pltpu.async_copy(
                    gather_src.at[gidx_v.at[pl.ds(b * 128, 128)]],
                    bufs[b][0], bufs[b][1])
            for j in range(n_chunks):
                rows_v, gsem, icur_v = bufs[j % 2]
                pltpu.make_async_copy(
                    gather_src.at[gidx_v.at[pl.ds(0, 128)]],
                    rows_v, gsem).wait()
                _copy_idx(icur_v, sidx_v, j * 128)
                pltpu.sync_copy(rows_v, acc.at[icur_v], add=True)
                if j + 2 < n_chunks:
                    pltpu.async_copy(
                        gather_src.at[gidx_v.at[pl.ds((j + 2) * 128, 128)]],
                        rows_v, gsem)

        _pipe(A_CHUNKS, xnew_hbm, aid_v, a2f_v, fragacc)
        plsc.subcore_barrier()

        _pipe(EF_CHUNKS, fragacc, fsrc_v, ftgt_v, ffs_acc)
        plsc.subcore_barrier()

        pltpu.sync_copy(ffs_acc.at[pl.ds(sid * F_STRIPE, F_STRIPE)],
                        ffs_hbm.at[pl.ds(sid * F_STRIPE, F_STRIPE)])


def _tc_mlp_body(ffs_ref, w1_ref, b1_ref, w2_ref, b2_ref, out_ref):
    h = jnp.dot(ffs_ref[:N_FRAGS], w1_ref[...],
                preferred_element_type=jnp.float32) + b1_ref[...]
    h = jnp.maximum(h, 0.0)
    out_ref[...] = jnp.dot(h, w2_ref[...],
                           preferred_element_type=jnp.float32) + b2_ref[...]


_tc_mlp = pl.pallas_call(
    _tc_mlp_body,
    out_shape=jax.ShapeDtypeStruct((N_FRAGS, D), jnp.float32),
)


def kernel(x_atoms, edge_index, edge_attr, frag_index, x_frags, atom_to_frag_ids,
           node_feautures_bond_graph, edge_index_bonds_graph, edge_attr_bond_graph,
           W_atom, b_atom, W_edge, b_edge, W_proj, b_proj, a_b,
           W_frag1, b_frag1, W_frag2, b_frag2):
    src = edge_index[0]
    tgt = edge_index[1]
    pad_idx = jnp.arange(E_PAD - E_ATOMS, dtype=jnp.int32)
    srcp = jnp.concatenate([src, N_ATOMS + pad_idx % (N_PAD - N_ATOMS)])
    tgtp = jnp.concatenate([tgt, N_ATOMS + pad_idx % (Z_ROWS - N_ATOMS)])

    counts = _sc_hist(srcp)
    y, y16, dinv = _tc_emb(x_atoms, W_atom, b_atom.reshape(1, D),
                           counts.reshape(NW, N_PAD).T)

    zeros_stripe = jnp.zeros((Z_STRIPE, D), jnp.float32)
    zeros16 = jnp.zeros((Z_STRIPE, D), jnp.bfloat16)
    zpart = _sc_edges(y16, srcp, tgtp, zeros16)
    x_atoms_new = _tc_fin(zpart, y, dinv)

    aidp = jnp.arange(N_PAD, dtype=jnp.int32) % N_ATOMS
    a2fp = jnp.pad(atom_to_frag_ids, (0, N_PAD - N_ATOMS),
                   constant_values=JUNK_F)
    fsrcp = jnp.pad(frag_index[0], (0, EF_PAD - E_FRAG), constant_values=JUNK_F)
    ftgtp = jnp.pad(frag_index[1], (0, EF_PAD - E_FRAG), constant_values=JUNK_F)

    ffs = _sc_frag(x_atoms_new, aidp, a2fp, fsrcp, ftgtp, zeros_stripe)
    x_frags_new = _tc_mlp(ffs, W_frag1, b_frag1.reshape(1, 2 * D),
                          W_frag2, b_frag2.reshape(1, D))
    return (x_atoms_new, x_frags_new)
